# baseline (speedup 1.0000x reference)
import functools

import jax
import jax.numpy as jnp
from jax import lax
from jax.experimental import pallas as pl
from jax.experimental.pallas import tpu as pltpu
from jax.experimental.pallas import tpu_sc as plsc

V, D, B, H = 1000000, 64, 4096, 200
LB = 128
NU = H * (B // LB)
VLAST = (V // LB - 1) * LB
VTAIL = (V // LB) * LB
SPW = 246


def _mesh_info():
    info = plsc.get_sparse_core_info()
    return info.num_cores, info.num_subcores


@functools.lru_cache(maxsize=None)
def _make_transpose():
    NC, NS = _mesh_info()
    mesh = plsc.VectorSubcoreMesh(core_axis_name="c", subcore_axis_name="s")

    @functools.partial(
        pl.kernel,
        mesh=mesh,
        out_type=jax.ShapeDtypeStruct((V // 2, 2 * D), jnp.float32),
        scratch_types=[
            pltpu.VMEM((D, LB), jnp.float32),
            pltpu.VMEM((D, LB), jnp.float32),
            pltpu.VMEM((D, LB), jnp.float32),
            pltpu.VMEM((D, LB), jnp.float32),
            pltpu.SemaphoreType.DMA,
            pltpu.SemaphoreType.DMA,
            pltpu.SemaphoreType.DMA,
            pltpu.SemaphoreType.DMA,
        ],
        compiler_params=pltpu.CompilerParams(needs_layout_passes=False),
    )
    def tk(tt_hbm, out_hbm, s0, s1, o0, o1, si0, si1, so0, so1):
        wid = lax.axis_index("s") * NC + lax.axis_index("c")
        slab = (s0, s1)
        outb = (o0, o1)
        si = (si0, si1)
        so = (so0, so1)

        def voff(t):
            return pl.multiple_of(
                jnp.minimum((wid * SPW + t) * LB, VLAST), LB)

        def start_in(t, s):
            pltpu.async_copy(tt_hbm.at[:, pl.ds(voff(t), LB)], slab[s], si[s])

        def wait_in(t, s):
            pltpu.make_async_copy(tt_hbm.at[:, pl.ds(voff(t), LB)], slab[s],
                                  si[s]).wait()

        def orow(t):
            return pl.multiple_of(lax.shift_right_logical(voff(t), 1), D)

        def start_out(t, s):
            pltpu.async_copy(outb[s], out_hbm.at[pl.ds(orow(t), D)], so[s])

        def wait_out(t, s):
            pltpu.make_async_copy(outb[s], out_hbm.at[pl.ds(orow(t), D)],
                                  so[s]).wait()

        iota = lax.iota(jnp.int32, 16)
        rowv = [iota + 16 * k for k in range(D // 16)]

        def shuffle(s):
            def body(i, carry):
                for half in range(2):
                    vloc = 2 * i + half
                    col = jnp.full((16,), 1, jnp.int32) * vloc
                    for k in range(D // 16):
                        vec = plsc.load_gather(slab[s], [rowv[k], col])
                        outb[s][i, pl.ds(half * D + k * 16, 16)] = vec
                return carry
            lax.fori_loop(0, D, body, 0)

        def unit(t, s, prefetch):
            wait_in(t, s)
            wait_out(t - 2, s)
            shuffle(s)
            start_out(t, s)
            if prefetch:
                start_in(t + 2, s)

        start_in(0, 0)
        start_in(1, 1)
        wait_in(0, 0)
        shuffle(0)
        start_out(0, 0)
        start_in(2, 0)
        wait_in(1, 1)
        shuffle(1)
        start_out(1, 1)
        start_in(3, 1)

        def body(i, carry):
            t = 2 + 2 * i
            unit(t, 0, True)
            unit(t + 1, 1, True)
            return carry

        lax.fori_loop(0, (SPW - 4) // 2, body, 0)

        unit(SPW - 2, 0, False)
        unit(SPW - 1, 1, False)
        wait_out(SPW - 2, 0)
        wait_out(SPW - 1, 1)

    return tk


@functools.lru_cache(maxsize=None)
def _make_gather():
    NC, NS = _mesh_info()
    NW = NC * NS
    upw = NU // NW
    EH = D // 8
    BH = B // LB
    mesh = plsc.VectorSubcoreMesh(core_axis_name="c", subcore_axis_name="s")

    @functools.partial(
        pl.kernel,
        mesh=mesh,
        out_type=jax.ShapeDtypeStruct((H, EH, BH, 8, LB), jnp.float32),
        scratch_types=[
            pltpu.VMEM((upw, LB), jnp.int32),
            pltpu.VMEM((LB + 64, D), jnp.float32),
            pltpu.VMEM((LB + 64, D), jnp.float32),
            pltpu.VMEM((EH, 8, LB), jnp.float32),
            pltpu.VMEM((EH, 8, LB), jnp.float32),
            pltpu.SemaphoreType.DMA,
            pltpu.SemaphoreType.DMA,
            pltpu.SemaphoreType.DMA,
            pltpu.SemaphoreType.DMA,
        ],
        compiler_params=pltpu.CompilerParams(use_tc_tiling_on_sc=False,
                                             needs_layout_passes=False),
    )
    def gk(tab_hbm, xt_hbm, tail_hbm, out_hbm, idx_all, r0, r1, o0, o1,
           sg0, sg1, so0, so1):
        wid = lax.axis_index("s") * NC + lax.axis_index("c")
        rows = (r0, r1)
        oblk = (o0, o1)
        sg = (sg0, sg1)
        so = (so0, so1)

        pltpu.sync_copy(xt_hbm.at[pl.ds(wid * upw, upw)], idx_all)
        pltpu.sync_copy(tail_hbm, r0.at[pl.ds(LB, 64)])
        pltpu.sync_copy(tail_hbm, r1.at[pl.ds(LB, 64)])

        def start_gather(t, s):
            pltpu.async_copy(tab_hbm.at[idx_all.at[t]],
                             rows[s].at[pl.ds(0, LB)], sg[s])

        def wait_gather(t, s):
            pltpu.make_async_copy(tab_hbm.at[idx_all.at[t]],
                                  rows[s].at[pl.ds(0, LB)], sg[s]).wait()

        def hb(t):
            u = wid * upw + t
            return lax.shift_right_logical(u, 5), lax.bitwise_and(u, 31)

        def start_out(t, s):
            h, bh = hb(t)
            for ehi in range(EH):
                pltpu.async_copy(oblk[s].at[ehi], out_hbm.at[h, ehi, bh],
                                 so[s])

        def wait_out(s):
            for ehi in range(EH):
                pltpu.make_async_copy(oblk[s].at[ehi], out_hbm.at[0, ehi, 0],
                                      so[s]).wait()

        iota = lax.iota(jnp.int32, 16)
        rowv = [iota + 16 * bg for bg in range(LB // 16)]

        def shuffle(s, rmap):
            def sh_body(ehi, carry):
                for elo in range(8):
                    e = ehi * 8 + elo
                    col = jnp.full((16,), 1, jnp.int32) * e
                    for bg in range(LB // 16):
                        v = plsc.load_gather(rows[s], [rmap[bg], col])
                        oblk[s][ehi, elo, pl.ds(bg * 16, 16)] = v
                return carry
            lax.fori_loop(0, EH, sh_body, 0)

        def make_rmap(t):
            rmap = []
            for bg in range(LB // 16):
                iv = idx_all[t, pl.ds(bg * 16, 16)]
                rmap.append(jnp.where(iv >= VTAIL, iv - (VTAIL - LB),
                                      rowv[bg]))
            return rmap

        def unit(t, s, prefetch):
            wait_gather(t, s)
            wait_out(s)
            shuffle(s, make_rmap(t))
            start_out(t, s)
            if prefetch:
                start_gather(t + 2, s)

        start_gather(0, 0)
        start_gather(1, 1)
        wait_gather(0, 0)
        shuffle(0, make_rmap(0))
        start_out(0, 0)
        start_gather(2, 0)
        wait_gather(1, 1)
        shuffle(1, make_rmap(1))
        start_out(1, 1)
        start_gather(3, 1)

        def body(i, carry):
            t = 2 + 2 * i
            unit(t, 0, True)
            unit(t + 1, 1, True)
            return carry

        lax.fori_loop(0, (upw - 4) // 2, body, 0)

        unit(upw - 2, 0, False)
        unit(upw - 1, 1, False)
        wait_out(0)
        wait_out(1)

    return gk


def kernel(X, embedding_matrix):
    lin = _make_transpose()(embedding_matrix.T)
    table_lin = lin.reshape(V, D)
    xt = X.T.astype(jnp.int32).reshape(NU, LB)
    tail = embedding_matrix[VTAIL:]
    out5 = _make_gather()(table_lin, xt, tail)
    return out5.transpose((2, 4, 0, 1, 3)).reshape(B, H, D)


# speedup vs baseline: 2.2592x; 2.2592x over previous
import functools

import jax
import jax.numpy as jnp
from jax import lax
from jax.experimental import pallas as pl
from jax.experimental.pallas import tpu as pltpu
from jax.experimental.pallas import tpu_sc as plsc

V, D, B, H = 1000000, 64, 4096, 200
LB = 128
NU = H * (B // LB)
VLAST = (V // LB - 1) * LB
VTAIL = (V // LB) * LB
SPW = 246


def _mesh_info():
    info = plsc.get_sparse_core_info()
    return info.num_cores, info.num_subcores


@functools.lru_cache(maxsize=None)
def _make_transpose():
    NC, NS = _mesh_info()
    mesh = plsc.VectorSubcoreMesh(core_axis_name="c", subcore_axis_name="s")

    @functools.partial(
        pl.kernel,
        mesh=mesh,
        out_type=jax.ShapeDtypeStruct((V // 2, 2 * D), jnp.float32),
        scratch_types=[
            pltpu.VMEM((D, LB), jnp.float32),
            pltpu.VMEM((D, LB), jnp.float32),
            pltpu.VMEM((D, LB), jnp.float32),
            pltpu.VMEM((D, LB), jnp.float32),
            pltpu.SemaphoreType.DMA,
            pltpu.SemaphoreType.DMA,
            pltpu.SemaphoreType.DMA,
            pltpu.SemaphoreType.DMA,
        ],
        compiler_params=pltpu.CompilerParams(needs_layout_passes=False),
    )
    def tk(tt_hbm, out_hbm, s0, s1, o0, o1, si0, si1, so0, so1):
        wid = lax.axis_index("s") * NC + lax.axis_index("c")
        slab = (s0, s1)
        outb = (o0, o1)
        si = (si0, si1)
        so = (so0, so1)

        def voff(t):
            return pl.multiple_of(
                jnp.minimum((wid * SPW + t) * LB, VLAST), LB)

        def start_in(t, s):
            pltpu.async_copy(tt_hbm.at[:, pl.ds(voff(t), LB)], slab[s], si[s])

        def wait_in(t, s):
            pltpu.make_async_copy(tt_hbm.at[:, pl.ds(voff(t), LB)], slab[s],
                                  si[s]).wait()

        def orow(t):
            return pl.multiple_of(lax.shift_right_logical(voff(t), 1), D)

        def start_out(t, s):
            pltpu.async_copy(outb[s], out_hbm.at[pl.ds(orow(t), D)], so[s])

        def wait_out(t, s):
            pltpu.make_async_copy(outb[s], out_hbm.at[pl.ds(orow(t), D)],
                                  so[s]).wait()

        iota = lax.iota(jnp.int32, 16)
        eoffc = [lax.bitwise_and(iota + g, 15) for g in range(16)]
        par64 = lax.bitwise_and(iota, 1) * D

        def shuffle(s):
            def body(vb, carry):
                v_idx = iota + vb * 16
                r_idx = lax.shift_right_logical(v_idx, 1)
                for k in range(D // 16):
                    for g in range(16):
                        e_idx = eoffc[g] + (k * 16)
                        val = plsc.load_gather(slab[s], [e_idx, v_idx])
                        plsc.store_scatter(outb[s], [r_idx, par64 + e_idx],
                                           val)
                return carry
            lax.fori_loop(0, LB // 16, body, 0)

        def unit(t, s, prefetch):
            wait_in(t, s)
            wait_out(t - 2, s)
            shuffle(s)
            start_out(t, s)
            if prefetch:
                start_in(t + 2, s)

        start_in(0, 0)
        start_in(1, 1)
        wait_in(0, 0)
        shuffle(0)
        start_out(0, 0)
        start_in(2, 0)
        wait_in(1, 1)
        shuffle(1)
        start_out(1, 1)
        start_in(3, 1)

        def body(i, carry):
            t = 2 + 2 * i
            unit(t, 0, True)
            unit(t + 1, 1, True)
            return carry

        lax.fori_loop(0, (SPW - 4) // 2, body, 0)

        unit(SPW - 2, 0, False)
        unit(SPW - 1, 1, False)
        wait_out(SPW - 2, 0)
        wait_out(SPW - 1, 1)

    return tk


@functools.lru_cache(maxsize=None)
def _make_gather():
    NC, NS = _mesh_info()
    NW = NC * NS
    upw = NU // NW
    EH = D // 8
    BH = B // LB
    mesh = plsc.VectorSubcoreMesh(core_axis_name="c", subcore_axis_name="s")

    @functools.partial(
        pl.kernel,
        mesh=mesh,
        out_type=jax.ShapeDtypeStruct((H, EH, BH, 8, LB), jnp.float32),
        scratch_types=[
            pltpu.VMEM((upw, LB), jnp.int32),
            pltpu.VMEM((LB + 64, D), jnp.float32),
            pltpu.VMEM((LB + 64, D), jnp.float32),
            pltpu.VMEM((D, LB), jnp.float32),
            pltpu.VMEM((D, LB), jnp.float32),
            pltpu.SemaphoreType.DMA,
            pltpu.SemaphoreType.DMA,
            pltpu.SemaphoreType.DMA,
            pltpu.SemaphoreType.DMA,
        ],
        compiler_params=pltpu.CompilerParams(use_tc_tiling_on_sc=False,
                                             needs_layout_passes=False),
    )
    def gk(tab_hbm, xt_hbm, tail_hbm, out_hbm, idx_all, r0, r1, o0, o1,
           sg0, sg1, so0, so1):
        wid = lax.axis_index("s") * NC + lax.axis_index("c")
        rows = (r0, r1)
        oblk = (o0, o1)
        sg = (sg0, sg1)
        so = (so0, so1)

        pltpu.sync_copy(xt_hbm.at[pl.ds(wid * upw, upw)], idx_all)
        pltpu.sync_copy(tail_hbm, r0.at[pl.ds(LB, 64)])
        pltpu.sync_copy(tail_hbm, r1.at[pl.ds(LB, 64)])

        def start_gather(t, s):
            pltpu.async_copy(tab_hbm.at[idx_all.at[t]],
                             rows[s].at[pl.ds(0, LB)], sg[s])

        def wait_gather(t, s):
            pltpu.make_async_copy(tab_hbm.at[idx_all.at[t]],
                                  rows[s].at[pl.ds(0, LB)], sg[s]).wait()

        def hb(t):
            u = wid * upw + t
            return lax.shift_right_logical(u, 5), lax.bitwise_and(u, 31)

        def start_out(t, s):
            h, bh = hb(t)
            for ehi in range(EH):
                pltpu.async_copy(oblk[s].at[pl.ds(ehi * 8, 8)],
                                 out_hbm.at[h, ehi, bh], so[s])

        def wait_out(s):
            for ehi in range(EH):
                pltpu.make_async_copy(oblk[s].at[pl.ds(ehi * 8, 8)],
                                      out_hbm.at[0, ehi, 0], so[s]).wait()

        iota = lax.iota(jnp.int32, 16)
        rowv = [iota + 16 * bg for bg in range(LB // 16)]
        eoffc = [lax.bitwise_and(iota + g, 15) for g in range(16)]

        def shuffle(s, rmap):
            def sh_body(k, carry):
                e0 = k * 16
                for bg in range(LB // 16):
                    for g in range(16):
                        e_idx = eoffc[g] + e0
                        val = plsc.load_gather(rows[s], [rmap[bg], e_idx])
                        plsc.store_scatter(oblk[s], [e_idx, rowv[bg]], val)
                return carry
            lax.fori_loop(0, D // 16, sh_body, 0)

        def make_rmap(t):
            rmap = []
            for bg in range(LB // 16):
                iv = idx_all[t, pl.ds(bg * 16, 16)]
                rmap.append(jnp.where(iv >= VTAIL, iv - (VTAIL - LB),
                                      rowv[bg]))
            return rmap

        def unit(t, s, prefetch):
            wait_gather(t, s)
            wait_out(s)
            shuffle(s, make_rmap(t))
            start_out(t, s)
            if prefetch:
                start_gather(t + 2, s)

        start_gather(0, 0)
        start_gather(1, 1)
        wait_gather(0, 0)
        shuffle(0, make_rmap(0))
        start_out(0, 0)
        start_gather(2, 0)
        wait_gather(1, 1)
        shuffle(1, make_rmap(1))
        start_out(1, 1)
        start_gather(3, 1)

        def body(i, carry):
            t = 2 + 2 * i
            unit(t, 0, True)
            unit(t + 1, 1, True)
            return carry

        lax.fori_loop(0, (upw - 4) // 2, body, 0)

        unit(upw - 2, 0, False)
        unit(upw - 1, 1, False)
        wait_out(0)
        wait_out(1)

    return gk


def kernel(X, embedding_matrix):
    lin = _make_transpose()(embedding_matrix.T)
    table_lin = lin.reshape(V, D)
    xt = X.T.astype(jnp.int32).reshape(NU, LB)
    tail = embedding_matrix[VTAIL:]
    out5 = _make_gather()(table_lin, xt, tail)
    return out5.transpose((2, 4, 0, 1, 3)).reshape(B, H, D)


# speedup vs baseline: 3.2599x; 1.4430x over previous
import functools

import jax
import jax.numpy as jnp
from jax import lax
from jax.experimental import pallas as pl
from jax.experimental.pallas import tpu as pltpu
from jax.experimental.pallas import tpu_sc as plsc

V, D, B, H = 1000000, 64, 4096, 200
LB = 128
NU = H * (B // LB)
VLAST = (V // LB - 1) * LB
VTAIL = (V // LB) * LB
SPW = 246


def _mesh_info():
    info = plsc.get_sparse_core_info()
    return info.num_cores, info.num_subcores


@functools.lru_cache(maxsize=None)
def _make_transpose():
    NC, NS = _mesh_info()
    mesh = plsc.VectorSubcoreMesh(core_axis_name="c", subcore_axis_name="s")

    @functools.partial(
        pl.kernel,
        mesh=mesh,
        out_type=jax.ShapeDtypeStruct((V // 2, 2 * D), jnp.float32),
        scratch_types=[
            pltpu.VMEM((D, LB), jnp.float32),
            pltpu.VMEM((D, LB), jnp.float32),
            pltpu.VMEM((D, LB), jnp.float32),
            pltpu.VMEM((D, LB), jnp.float32),
            pltpu.SemaphoreType.DMA,
            pltpu.SemaphoreType.DMA,
            pltpu.SemaphoreType.DMA,
            pltpu.SemaphoreType.DMA,
        ],
        compiler_params=pltpu.CompilerParams(needs_layout_passes=False,
                                             disable_bounds_checks=True),
    )
    def tk(tt_hbm, out_hbm, s0, s1, o0, o1, si0, si1, so0, so1):
        wid = lax.axis_index("s") * NC + lax.axis_index("c")
        slab = (s0, s1)
        outb = (o0, o1)
        si = (si0, si1)
        so = (so0, so1)

        def voff(t):
            return pl.multiple_of(
                jnp.minimum((wid * SPW + t) * LB, VLAST), LB)

        def start_in(t, s):
            pltpu.async_copy(tt_hbm.at[:, pl.ds(voff(t), LB)], slab[s], si[s])

        def wait_in(t, s):
            pltpu.make_async_copy(tt_hbm.at[:, pl.ds(voff(t), LB)], slab[s],
                                  si[s]).wait()

        def orow(t):
            return pl.multiple_of(lax.shift_right_logical(voff(t), 1), D)

        def start_out(t, s):
            pltpu.async_copy(outb[s], out_hbm.at[pl.ds(orow(t), D)], so[s])

        def wait_out(t, s):
            pltpu.make_async_copy(outb[s], out_hbm.at[pl.ds(orow(t), D)],
                                  so[s]).wait()

        iota = lax.iota(jnp.int32, 16)
        eoffc = [lax.bitwise_and(iota + g, 15) for g in range(16)]
        par64 = lax.bitwise_and(iota, 1) * D

        def shuffle(s):
            @plsc.parallel_loop(0, LB // 16, step=1, unroll=2)
            def body(vb):
                v_idx = iota + vb * 16
                r_idx = lax.shift_right_logical(v_idx, 1)
                for k in range(D // 16):
                    for g in range(16):
                        e_idx = eoffc[g] + (k * 16)
                        val = plsc.load_gather(slab[s], [e_idx, v_idx])
                        plsc.store_scatter(outb[s], [r_idx, par64 + e_idx],
                                           val)

        def unit(t, s, prefetch):
            wait_in(t, s)
            wait_out(t - 2, s)
            shuffle(s)
            start_out(t, s)
            if prefetch:
                start_in(t + 2, s)

        start_in(0, 0)
        start_in(1, 1)
        wait_in(0, 0)
        shuffle(0)
        start_out(0, 0)
        start_in(2, 0)
        wait_in(1, 1)
        shuffle(1)
        start_out(1, 1)
        start_in(3, 1)

        def body(i, carry):
            t = 2 + 2 * i
            unit(t, 0, True)
            unit(t + 1, 1, True)
            return carry

        lax.fori_loop(0, (SPW - 4) // 2, body, 0)

        unit(SPW - 2, 0, False)
        unit(SPW - 1, 1, False)
        wait_out(SPW - 2, 0)
        wait_out(SPW - 1, 1)

    return tk


@functools.lru_cache(maxsize=None)
def _make_gather():
    NC, NS = _mesh_info()
    NW = NC * NS
    upw = NU // NW
    EH = D // 8
    BH = B // LB
    mesh = plsc.VectorSubcoreMesh(core_axis_name="c", subcore_axis_name="s")

    @functools.partial(
        pl.kernel,
        mesh=mesh,
        out_type=jax.ShapeDtypeStruct((H, EH, BH, 8, LB), jnp.float32),
        scratch_types=[
            pltpu.VMEM((upw, LB), jnp.int32),
            pltpu.VMEM((LB + 64, D), jnp.float32),
            pltpu.VMEM((LB + 64, D), jnp.float32),
            pltpu.VMEM((D, LB), jnp.float32),
            pltpu.VMEM((D, LB), jnp.float32),
            pltpu.SemaphoreType.DMA,
            pltpu.SemaphoreType.DMA,
            pltpu.SemaphoreType.DMA,
            pltpu.SemaphoreType.DMA,
        ],
        compiler_params=pltpu.CompilerParams(use_tc_tiling_on_sc=False,
                                             needs_layout_passes=False,
                                             disable_bounds_checks=True),
    )
    def gk(tab_hbm, xt_hbm, tail_hbm, out_hbm, idx_all, r0, r1, o0, o1,
           sg0, sg1, so0, so1):
        wid = lax.axis_index("s") * NC + lax.axis_index("c")
        rows = (r0, r1)
        oblk = (o0, o1)
        sg = (sg0, sg1)
        so = (so0, so1)

        pltpu.sync_copy(xt_hbm.at[pl.ds(wid * upw, upw)], idx_all)
        pltpu.sync_copy(tail_hbm, r0.at[pl.ds(LB, 64)])
        pltpu.sync_copy(tail_hbm, r1.at[pl.ds(LB, 64)])

        def start_gather(t, s):
            pltpu.async_copy(tab_hbm.at[idx_all.at[t]],
                             rows[s].at[pl.ds(0, LB)], sg[s])

        def wait_gather(t, s):
            pltpu.make_async_copy(tab_hbm.at[idx_all.at[t]],
                                  rows[s].at[pl.ds(0, LB)], sg[s]).wait()

        def hb(t):
            u = wid * upw + t
            return lax.shift_right_logical(u, 5), lax.bitwise_and(u, 31)

        def start_out(t, s):
            h, bh = hb(t)
            for ehi in range(EH):
                pltpu.async_copy(oblk[s].at[pl.ds(ehi * 8, 8)],
                                 out_hbm.at[h, ehi, bh], so[s])

        def wait_out(s):
            for ehi in range(EH):
                pltpu.make_async_copy(oblk[s].at[pl.ds(ehi * 8, 8)],
                                      out_hbm.at[0, ehi, 0], so[s]).wait()

        iota = lax.iota(jnp.int32, 16)
        rowv = [iota + 16 * bg for bg in range(LB // 16)]
        eoffc = [lax.bitwise_and(iota + g, 15) for g in range(16)]

        def shuffle(s, rmap):
            @plsc.parallel_loop(0, D // 16, step=1, unroll=2)
            def sh_body(k):
                e0 = k * 16
                for bg in range(LB // 16):
                    for g in range(16):
                        e_idx = eoffc[g] + e0
                        val = plsc.load_gather(rows[s], [rmap[bg], e_idx])
                        plsc.store_scatter(oblk[s], [e_idx, rowv[bg]], val)

        def make_rmap(t):
            rmap = []
            for bg in range(LB // 16):
                iv = idx_all[t, pl.ds(bg * 16, 16)]
                rmap.append(jnp.where(iv >= VTAIL, iv - (VTAIL - LB),
                                      rowv[bg]))
            return rmap

        def unit(t, s, prefetch):
            wait_gather(t, s)
            wait_out(s)
            shuffle(s, make_rmap(t))
            start_out(t, s)
            if prefetch:
                start_gather(t + 2, s)

        start_gather(0, 0)
        start_gather(1, 1)
        wait_gather(0, 0)
        shuffle(0, make_rmap(0))
        start_out(0, 0)
        start_gather(2, 0)
        wait_gather(1, 1)
        shuffle(1, make_rmap(1))
        start_out(1, 1)
        start_gather(3, 1)

        def body(i, carry):
            t = 2 + 2 * i
            unit(t, 0, True)
            unit(t + 1, 1, True)
            return carry

        lax.fori_loop(0, (upw - 4) // 2, body, 0)

        unit(upw - 2, 0, False)
        unit(upw - 1, 1, False)
        wait_out(0)
        wait_out(1)

    return gk


def kernel(X, embedding_matrix):
    lin = _make_transpose()(embedding_matrix.T)
    table_lin = lin.reshape(V, D)
    xt = X.T.astype(jnp.int32).reshape(NU, LB)
    tail = embedding_matrix[VTAIL:]
    out5 = _make_gather()(table_lin, xt, tail)
    return out5.transpose((2, 4, 0, 1, 3)).reshape(B, H, D)


# speedup vs baseline: 4.0140x; 1.2313x over previous
import functools

import jax
import jax.numpy as jnp
from jax import lax
from jax.experimental import pallas as pl
from jax.experimental.pallas import tpu as pltpu
from jax.experimental.pallas import tpu_sc as plsc

V, D, B, H = 1000000, 64, 4096, 200
LB = 128
NU = H * (B // LB)
VLAST = (V // LB - 1) * LB
VTAIL = (V // LB) * LB
SPW = 246


def _mesh_info():
    info = plsc.get_sparse_core_info()
    return info.num_cores, info.num_subcores


@functools.lru_cache(maxsize=None)
def _make_transpose():
    NC, NS = _mesh_info()
    mesh = plsc.VectorSubcoreMesh(core_axis_name="c", subcore_axis_name="s")

    @functools.partial(
        pl.kernel,
        mesh=mesh,
        out_type=jax.ShapeDtypeStruct((V // 2, 2 * D), jnp.float32),
        scratch_types=[
            pltpu.VMEM((D, LB), jnp.float32),
            pltpu.VMEM((D, LB), jnp.float32),
            pltpu.VMEM((D, LB), jnp.float32),
            pltpu.VMEM((D, LB), jnp.float32),
            pltpu.SemaphoreType.DMA,
            pltpu.SemaphoreType.DMA,
            pltpu.SemaphoreType.DMA,
            pltpu.SemaphoreType.DMA,
        ],
        compiler_params=pltpu.CompilerParams(needs_layout_passes=False,
                                             disable_bounds_checks=True),
    )
    def tk(tt_hbm, out_hbm, s0, s1, o0, o1, si0, si1, so0, so1):
        wid = lax.axis_index("s") * NC + lax.axis_index("c")
        slab = (s0, s1)
        outb = (o0, o1)
        si = (si0, si1)
        so = (so0, so1)

        def voff(t):
            return pl.multiple_of(
                jnp.minimum((wid * SPW + t) * LB, VLAST), LB)

        def start_in(t, s):
            pltpu.async_copy(tt_hbm.at[:, pl.ds(voff(t), LB)], slab[s], si[s])

        def wait_in(t, s):
            pltpu.make_async_copy(tt_hbm.at[:, pl.ds(voff(t), LB)], slab[s],
                                  si[s]).wait()

        def orow(t):
            return pl.multiple_of(lax.shift_right_logical(voff(t), 1), D)

        def start_out(t, s):
            pltpu.async_copy(outb[s], out_hbm.at[pl.ds(orow(t), D)], so[s])

        def wait_out(t, s):
            pltpu.make_async_copy(outb[s], out_hbm.at[pl.ds(orow(t), D)],
                                  so[s]).wait()

        iota = lax.iota(jnp.int32, 16)
        eoffc = [lax.bitwise_and(iota + g, 15) for g in range(16)]
        par64 = lax.bitwise_and(iota, 1) * D

        def shuffle(s):
            @plsc.parallel_loop(0, LB // 16, step=1, unroll=2)
            def body(vb):
                v_idx = iota + vb * 16
                r_idx = lax.shift_right_logical(v_idx, 1)
                for k in range(D // 16):
                    for g in range(16):
                        e_idx = eoffc[g] + (k * 16)
                        val = plsc.load_gather(slab[s], [e_idx, v_idx])
                        plsc.store_scatter(outb[s], [r_idx, par64 + e_idx],
                                           val)

        def unit(t, s, prefetch):
            wait_in(t, s)
            wait_out(t - 2, s)
            shuffle(s)
            start_out(t, s)
            if prefetch:
                start_in(t + 2, s)

        start_in(0, 0)
        start_in(1, 1)
        wait_in(0, 0)
        shuffle(0)
        start_out(0, 0)
        start_in(2, 0)
        wait_in(1, 1)
        shuffle(1)
        start_out(1, 1)
        start_in(3, 1)

        def body(i, carry):
            t = 2 + 2 * i
            unit(t, 0, True)
            unit(t + 1, 1, True)
            return carry

        lax.fori_loop(0, (SPW - 4) // 2, body, 0)

        unit(SPW - 2, 0, False)
        unit(SPW - 1, 1, False)
        wait_out(SPW - 2, 0)
        wait_out(SPW - 1, 1)

    return tk


@functools.lru_cache(maxsize=None)
def _make_gather():
    NC, NS = _mesh_info()
    NW = NC * NS
    upw = NU // NW
    EH = D // 8
    BH = B // LB
    mesh = plsc.VectorSubcoreMesh(core_axis_name="c", subcore_axis_name="s")

    @functools.partial(
        pl.kernel,
        mesh=mesh,
        out_type=jax.ShapeDtypeStruct((H, EH, BH, 8, LB), jnp.float32),
        scratch_types=[
            pltpu.VMEM((upw, LB), jnp.int32),
            pltpu.VMEM((LB + 64, D), jnp.float32),
            pltpu.VMEM((LB + 64, D), jnp.float32),
            pltpu.VMEM((D, LB), jnp.float32),
            pltpu.VMEM((D, LB), jnp.float32),
            pltpu.SemaphoreType.DMA,
            pltpu.SemaphoreType.DMA,
            pltpu.SemaphoreType.DMA,
            pltpu.SemaphoreType.DMA,
        ],
        compiler_params=pltpu.CompilerParams(use_tc_tiling_on_sc=False,
                                             needs_layout_passes=False,
                                             disable_bounds_checks=True),
    )
    def gk(tab_hbm, xt_hbm, tail_hbm, out_hbm, idx_all, r0, r1, o0, o1,
           sg0, sg1, so0, so1):
        wid = lax.axis_index("s") * NC + lax.axis_index("c")
        rows = (r0, r1)
        oblk = (o0, o1)
        sg = (sg0, sg1)
        so = (so0, so1)

        pltpu.sync_copy(xt_hbm.at[pl.ds(wid * upw, upw)], idx_all)
        pltpu.sync_copy(tail_hbm, r0.at[pl.ds(LB, 64)])
        pltpu.sync_copy(tail_hbm, r1.at[pl.ds(LB, 64)])

        def start_gather(t, s):
            pltpu.async_copy(tab_hbm.at[idx_all.at[t]],
                             rows[s].at[pl.ds(0, LB)], sg[s])

        def wait_gather(t, s):
            pltpu.make_async_copy(tab_hbm.at[idx_all.at[t]],
                                  rows[s].at[pl.ds(0, LB)], sg[s]).wait()

        def hb(t):
            u = wid * upw + t
            return lax.shift_right_logical(u, 5), lax.bitwise_and(u, 31)

        def start_out(t, s):
            h, bh = hb(t)
            for ehi in range(EH):
                pltpu.async_copy(oblk[s].at[pl.ds(ehi * 8, 8)],
                                 out_hbm.at[h, ehi, bh], so[s])

        def wait_out(s):
            for ehi in range(EH):
                pltpu.make_async_copy(oblk[s].at[pl.ds(ehi * 8, 8)],
                                      out_hbm.at[0, ehi, 0], so[s]).wait()

        iota = lax.iota(jnp.int32, 16)
        rowv = [iota + 16 * bg for bg in range(LB // 16)]
        eoffc = [lax.bitwise_and(iota + g, 15) for g in range(16)]

        def shuffle(s, rmap):
            @plsc.parallel_loop(0, D // 16, step=1, unroll=2)
            def sh_body(k):
                e0 = k * 16
                for g in range(16):
                    e_idx = eoffc[g] + e0
                    for bg in range(LB // 16):
                        val = plsc.load_gather(rows[s], [rmap[bg], e_idx])
                        plsc.store_scatter(oblk[s], [e_idx, rowv[bg]], val)

        def make_rmap(t):
            rmap = []
            for bg in range(LB // 16):
                iv = idx_all[t, pl.ds(bg * 16, 16)]
                rmap.append(jnp.where(iv >= VTAIL, iv - (VTAIL - LB),
                                      rowv[bg]))
            return rmap

        def unit(t, s, prefetch):
            wait_gather(t, s)
            wait_out(s)
            shuffle(s, make_rmap(t))
            start_out(t, s)
            if prefetch:
                start_gather(t + 2, s)

        start_gather(0, 0)
        start_gather(1, 1)
        wait_gather(0, 0)
        shuffle(0, make_rmap(0))
        start_out(0, 0)
        start_gather(2, 0)
        wait_gather(1, 1)
        shuffle(1, make_rmap(1))
        start_out(1, 1)
        start_gather(3, 1)

        def body(i, carry):
            t = 2 + 2 * i
            unit(t, 0, True)
            unit(t + 1, 1, True)
            return carry

        lax.fori_loop(0, (upw - 4) // 2, body, 0)

        unit(upw - 2, 0, False)
        unit(upw - 1, 1, False)
        wait_out(0)
        wait_out(1)

    return gk


def kernel(X, embedding_matrix):
    lin = _make_transpose()(embedding_matrix.T)
    table_lin = lin.reshape(V, D)
    xt = X.T.astype(jnp.int32).reshape(NU, LB)
    tail = embedding_matrix[VTAIL:]
    out5 = _make_gather()(table_lin, xt, tail)
    return out5.transpose((2, 4, 0, 1, 3)).reshape(B, H, D)


# speedup vs baseline: 5.0223x; 1.2512x over previous
import functools

import jax
import jax.numpy as jnp
from jax import lax
from jax.experimental import pallas as pl
from jax.experimental.pallas import tpu as pltpu
from jax.experimental.pallas import tpu_sc as plsc

V, D, B, H = 1000000, 64, 4096, 200
LB = 128
NU = H * (B // LB)
VLAST = (V // LB - 1) * LB
VTAIL = (V // LB) * LB
SPW = 246


def _mesh_info():
    info = plsc.get_sparse_core_info()
    return info.num_cores, info.num_subcores


@functools.lru_cache(maxsize=None)
def _make_transpose():
    NC, NS = _mesh_info()
    mesh = plsc.VectorSubcoreMesh(core_axis_name="c", subcore_axis_name="s")

    @functools.partial(
        pl.kernel,
        mesh=mesh,
        out_type=jax.ShapeDtypeStruct((V // 2, 2 * D), jnp.float32),
        scratch_types=[
            pltpu.VMEM((D, LB), jnp.float32),
            pltpu.VMEM((D, LB), jnp.float32),
            pltpu.VMEM((D, LB), jnp.float32),
            pltpu.VMEM((D, LB), jnp.float32),
            pltpu.SemaphoreType.DMA,
            pltpu.SemaphoreType.DMA,
            pltpu.SemaphoreType.DMA,
            pltpu.SemaphoreType.DMA,
        ],
        compiler_params=pltpu.CompilerParams(needs_layout_passes=False,
                                             disable_bounds_checks=True),
    )
    def tk(tt_hbm, out_hbm, s0, s1, o0, o1, si0, si1, so0, so1):
        wid = lax.axis_index("s") * NC + lax.axis_index("c")
        slab = (s0, s1)
        outb = (o0, o1)
        si = (si0, si1)
        so = (so0, so1)

        def voff(t):
            return pl.multiple_of(
                jnp.minimum((wid * SPW + t) * LB, VLAST), LB)

        def start_in(t, s):
            pltpu.async_copy(tt_hbm.at[:, pl.ds(voff(t), LB)], slab[s], si[s])

        def wait_in(t, s):
            pltpu.make_async_copy(tt_hbm.at[:, pl.ds(voff(t), LB)], slab[s],
                                  si[s]).wait()

        def orow(t):
            return pl.multiple_of(lax.shift_right_logical(voff(t), 1), D)

        def start_out(t, s):
            pltpu.async_copy(outb[s], out_hbm.at[pl.ds(orow(t), D)], so[s])

        def wait_out(t, s):
            pltpu.make_async_copy(outb[s], out_hbm.at[pl.ds(orow(t), D)],
                                  so[s]).wait()

        iota = lax.iota(jnp.int32, 16)
        eoffc = [lax.bitwise_and(iota + g, 15) for g in range(16)]
        par64 = lax.bitwise_and(iota, 1) * D
        vix = [iota + 16 * vb for vb in range(LB // 16)]
        rix = [lax.shift_right_logical(v, 1) for v in vix]

        def shuffle(s):
            @plsc.parallel_loop(0, D // 16, step=1, unroll=2)
            def body(k):
                e0 = k * 16
                for g in range(16):
                    e_idx = eoffc[g] + e0
                    c_idx = par64 + e_idx
                    for vb in range(LB // 16):
                        val = plsc.load_gather(slab[s], [e_idx, vix[vb]])
                        plsc.store_scatter(outb[s], [rix[vb], c_idx], val)

        def unit(t, s, prefetch):
            wait_in(t, s)
            wait_out(t - 2, s)
            shuffle(s)
            start_out(t, s)
            if prefetch:
                start_in(t + 2, s)

        start_in(0, 0)
        start_in(1, 1)
        wait_in(0, 0)
        shuffle(0)
        start_out(0, 0)
        start_in(2, 0)
        wait_in(1, 1)
        shuffle(1)
        start_out(1, 1)
        start_in(3, 1)

        def body(i, carry):
            t = 2 + 2 * i
            unit(t, 0, True)
            unit(t + 1, 1, True)
            return carry

        lax.fori_loop(0, (SPW - 4) // 2, body, 0)

        unit(SPW - 2, 0, False)
        unit(SPW - 1, 1, False)
        wait_out(SPW - 2, 0)
        wait_out(SPW - 1, 1)

    return tk


@functools.lru_cache(maxsize=None)
def _make_gather():
    NC, NS = _mesh_info()
    NW = NC * NS
    upw = NU // NW
    EH = D // 8
    BH = B // LB
    mesh = plsc.VectorSubcoreMesh(core_axis_name="c", subcore_axis_name="s")

    @functools.partial(
        pl.kernel,
        mesh=mesh,
        out_type=jax.ShapeDtypeStruct((H, EH, BH, 8, LB), jnp.float32),
        scratch_types=[
            pltpu.VMEM((upw, LB), jnp.int32),
            pltpu.VMEM((LB + 64, D), jnp.float32),
            pltpu.VMEM((LB + 64, D), jnp.float32),
            pltpu.VMEM((D, LB), jnp.float32),
            pltpu.VMEM((D, LB), jnp.float32),
            pltpu.SemaphoreType.DMA,
            pltpu.SemaphoreType.DMA,
            pltpu.SemaphoreType.DMA,
            pltpu.SemaphoreType.DMA,
        ],
        compiler_params=pltpu.CompilerParams(use_tc_tiling_on_sc=False,
                                             needs_layout_passes=False,
                                             disable_bounds_checks=True),
    )
    def gk(tab_hbm, xt_hbm, tail_hbm, out_hbm, idx_all, r0, r1, o0, o1,
           sg0, sg1, so0, so1):
        wid = lax.axis_index("s") * NC + lax.axis_index("c")
        rows = (r0, r1)
        oblk = (o0, o1)
        sg = (sg0, sg1)
        so = (so0, so1)

        pltpu.sync_copy(xt_hbm.at[pl.ds(wid * upw, upw)], idx_all)
        pltpu.sync_copy(tail_hbm, r0.at[pl.ds(LB, 64)])
        pltpu.sync_copy(tail_hbm, r1.at[pl.ds(LB, 64)])

        def start_gather(t, s):
            pltpu.async_copy(tab_hbm.at[idx_all.at[t]],
                             rows[s].at[pl.ds(0, LB)], sg[s])

        def wait_gather(t, s):
            pltpu.make_async_copy(tab_hbm.at[idx_all.at[t]],
                                  rows[s].at[pl.ds(0, LB)], sg[s]).wait()

        def hb(t):
            u = wid * upw + t
            return lax.shift_right_logical(u, 5), lax.bitwise_and(u, 31)

        def start_out(t, s):
            h, bh = hb(t)
            for ehi in range(EH):
                pltpu.async_copy(oblk[s].at[pl.ds(ehi * 8, 8)],
                                 out_hbm.at[h, ehi, bh], so[s])

        def wait_out(s):
            for ehi in range(EH):
                pltpu.make_async_copy(oblk[s].at[pl.ds(ehi * 8, 8)],
                                      out_hbm.at[0, ehi, 0], so[s]).wait()

        iota = lax.iota(jnp.int32, 16)
        rowv = [iota + 16 * bg for bg in range(LB // 16)]
        eoffc = [lax.bitwise_and(iota + g, 15) for g in range(16)]

        def shuffle(s, rmap):
            @plsc.parallel_loop(0, D // 16, step=1, unroll=2)
            def sh_body(k):
                e0 = k * 16
                for g in range(16):
                    e_idx = eoffc[g] + e0
                    for bg in range(LB // 16):
                        val = plsc.load_gather(rows[s], [rmap[bg], e_idx])
                        plsc.store_scatter(oblk[s], [e_idx, rowv[bg]], val)

        def make_rmap(t):
            rmap = []
            for bg in range(LB // 16):
                iv = idx_all[t, pl.ds(bg * 16, 16)]
                rmap.append(jnp.where(iv >= VTAIL, iv - (VTAIL - LB),
                                      rowv[bg]))
            return rmap

        def unit(t, s, prefetch):
            wait_gather(t, s)
            wait_out(s)
            shuffle(s, make_rmap(t))
            start_out(t, s)
            if prefetch:
                start_gather(t + 2, s)

        start_gather(0, 0)
        start_gather(1, 1)
        wait_gather(0, 0)
        shuffle(0, make_rmap(0))
        start_out(0, 0)
        start_gather(2, 0)
        wait_gather(1, 1)
        shuffle(1, make_rmap(1))
        start_out(1, 1)
        start_gather(3, 1)

        def body(i, carry):
            t = 2 + 2 * i
            unit(t, 0, True)
            unit(t + 1, 1, True)
            return carry

        lax.fori_loop(0, (upw - 4) // 2, body, 0)

        unit(upw - 2, 0, False)
        unit(upw - 1, 1, False)
        wait_out(0)
        wait_out(1)

    return gk


def kernel(X, embedding_matrix):
    lin = _make_transpose()(embedding_matrix.T)
    table_lin = lin.reshape(V, D)
    xt = X.T.astype(jnp.int32).reshape(NU, LB)
    tail = embedding_matrix[VTAIL:]
    out5 = _make_gather()(table_lin, xt, tail)
    return out5.transpose((2, 4, 0, 1, 3)).reshape(B, H, D)
